# baseline (speedup 1.0000x reference)
import functools

import jax
import jax.numpy as jnp
from jax import lax
from jax.experimental import pallas as pl
from jax.experimental.pallas import tpu as pltpu
from jax.experimental.pallas import tpu_sc as plsc

DOMAIN_MIN = -2.0
DOMAIN_MAX = 2.0


def _slopes_from_yt(yt, h, K):
    delta = (yt[1:] - yt[:-1]) * (1.0 / h)
    d0 = delta[:-1]
    d1 = delta[1:]
    same = (d0 * d1) > 0
    denom = d0 + d1
    small = jnp.abs(denom) < 1e-12
    safe_denom = jnp.where(small, jnp.ones_like(denom), denom)
    hm = 2.0 * d0 * d1 / safe_denom
    hm = jnp.where(small, jnp.zeros_like(hm), hm)
    mid = jnp.where(same, hm, jnp.zeros_like(hm))
    m0 = (3.0 * delta[0] - delta[1]) / 2.0
    mN = (3.0 * delta[-1] - delta[-2]) / 2.0
    m0 = jnp.where(m0 * delta[0] <= 0, jnp.zeros_like(m0), m0)
    mN = jnp.where(mN * delta[-1] <= 0, jnp.zeros_like(mN), mN)
    cond0 = (delta[0] * delta[1] < 0) & (jnp.abs(m0) > jnp.abs(3.0 * delta[0]))
    m0 = jnp.where(cond0, 3.0 * delta[0], m0)
    condN = (delta[-1] * delta[-2] < 0) & (jnp.abs(mN) > jnp.abs(3.0 * delta[-1]))
    mN = jnp.where(condN, 3.0 * delta[-1], mN)
    return m0, mid, mN


def _builder_kernel(yt_ref, ty_ref, tm_ref, *, K, h, oh):
    yt = yt_ref[...]
    m0, mid, mN = _slopes_from_yt(yt, h, K)
    mks = [h * m0] + [h * mid[k - 1] for k in range(1, K - 1)] + [h * mN]
    for s in range(2):
        sl = slice(s * oh, (s + 1) * oh)
        for k in range(K):
            c = (k % 4) * oh
            ty_ref[s, :, k // 4, c:c + oh] = yt[k][:, sl]
            tm_ref[s, :, k // 4, c:c + oh] = mks[k][:, sl]


def _build_tables(yt, K, d_in, d_out, h):
    oh = d_out // 2
    return pl.pallas_call(
        functools.partial(_builder_kernel, K=K, h=h, oh=oh),
        out_shape=(
            jax.ShapeDtypeStruct((2, d_in, K // 4, 4 * oh), jnp.float32),
            jax.ShapeDtypeStruct((2, d_in, K // 4, 4 * oh), jnp.float32),
        ),
    )(yt)


def _sc_interp_call(xs, ty, tm, bias, *, K, d_in, d_out, h):
    B_sc = xs.shape[0]
    NC, NS = 2, 16
    NW = NC * NS
    n_bchunk = NW // 2
    bc = B_sc // n_bchunk
    oh = d_out // 2
    inv_h = 1.0 / h
    nrow = d_in * K // 4
    mesh = plsc.VectorSubcoreMesh(core_axis_name="c", subcore_axis_name="s")

    x2 = xs.reshape(B_sc // 2, 2 * d_in)
    ty2 = ty.reshape(2, nrow, 128)
    tm2 = tm.reshape(2, nrow, 128)

    @functools.partial(
        pl.kernel,
        mesh=mesh,
        out_type=jax.ShapeDtypeStruct((2, B_sc // 4, 4 * oh), jnp.float32),
        scratch_types=[
            pltpu.VMEM((bc // 2, 2 * d_in), jnp.float32),
            pltpu.VMEM((nrow, 128), jnp.float32),
            pltpu.VMEM((nrow, 128), jnp.float32),
            pltpu.VMEM((oh,), jnp.float32),
            pltpu.VMEM((bc // 4, 4 * oh), jnp.float32),
        ],
    )
    def sc_kernel(x_hbm, ty_hbm, tm_hbm, bias_hbm, out_hbm,
                  x_v, ty_v, tm_v, bias_v, out_v):
        wid = lax.axis_index("s") * NC + lax.axis_index("c")
        bchunk = wid // 2
        ohalf = wid % 2
        pltpu.sync_copy(x_hbm.at[pl.ds(bchunk * (bc // 2), bc // 2), :], x_v)
        pltpu.sync_copy(ty_hbm.at[ohalf], ty_v)
        pltpu.sync_copy(tm_hbm.at[ohalf], tm_v)
        pltpu.sync_copy(bias_hbm.at[ohalf], bias_v)

        def sample_body(b, _):
            aa = bias_v[pl.ds(0, 16)]
            ab = bias_v[pl.ds(16, 16)]
            xrow = b >> 1
            xcol = (b & 1) * d_in
            for blk in range(d_in // 16):
                xv = x_v[xrow, pl.ds(xcol + blk * 16, 16)]
                xc = jnp.minimum(jnp.maximum(xv, DOMAIN_MIN), DOMAIN_MAX)
                t = (xc - DOMAIN_MIN) * inv_h
                idx = jnp.minimum(t.astype(jnp.int32), K - 2)
                u = t - idx.astype(jnp.float32)
                u2 = u * u
                u3 = u2 * u
                h00v = 2.0 * u3 - 3.0 * u2 + 1.0
                h10v = u3 - 2.0 * u2 + u
                h01v = -2.0 * u3 + 3.0 * u2
                h11v = u3 - u2
                for l in range(16):
                    i = blk * 16 + l
                    k0 = idx[l]
                    k1 = k0 + 1
                    h00 = h00v[l]
                    h10 = h10v[l]
                    h01 = h01v[l]
                    h11 = h11v[l]
                    r0 = i * 4 + (k0 >> 2)
                    c0 = (k0 & 3) * oh
                    r1 = i * 4 + (k1 >> 2)
                    c1 = (k1 & 3) * oh
                    y0a = ty_v[r0, pl.ds(c0, 16)]
                    y0b = ty_v[r0, pl.ds(c0 + 16, 16)]
                    y1a = ty_v[r1, pl.ds(c1, 16)]
                    y1b = ty_v[r1, pl.ds(c1 + 16, 16)]
                    m0a = tm_v[r0, pl.ds(c0, 16)]
                    m0b = tm_v[r0, pl.ds(c0 + 16, 16)]
                    m1a = tm_v[r1, pl.ds(c1, 16)]
                    m1b = tm_v[r1, pl.ds(c1 + 16, 16)]
                    aa = aa + h00 * y0a + h01 * y1a + h10 * m0a + h11 * m1a
                    ab = ab + h00 * y0b + h01 * y1b + h10 * m0b + h11 * m1b
            orow = b >> 2
            ocol = (b & 3) * oh
            out_v[orow, pl.ds(ocol, 16)] = aa
            out_v[orow, pl.ds(ocol + 16, 16)] = ab
            return 0

        lax.fori_loop(0, bc, sample_body, 0)
        pltpu.sync_copy(out_v, out_hbm.at[ohalf, pl.ds(bchunk * (bc // 4), bc // 4), :])

    out2 = sc_kernel(x2, ty2, tm2, bias.reshape(2, oh))
    return jnp.concatenate(
        [out2[0].reshape(B_sc, oh), out2[1].reshape(B_sc, oh)], axis=1)


def _tc_matmul_kernel(x_ref, yt_ref, bias_ref, out_ref, a_ref, *, K, d_in, d_out, h):
    @pl.when(pl.program_id(0) == 0)
    def _build_table():
        yt = yt_ref[...]
        m0, mid, mN = _slopes_from_yt(yt, h, K)
        a_ref[0:d_in, :] = yt[0]
        a_ref[(K - 1) * d_in:K * d_in, :] = yt[K - 1]
        a_ref[K * d_in:(K + 1) * d_in, :] = h * m0
        a_ref[(2 * K - 1) * d_in:2 * K * d_in, :] = h * mN
        for k in range(1, K - 1):
            a_ref[k * d_in:(k + 1) * d_in, :] = yt[k]
            a_ref[(K + k) * d_in:(K + k + 1) * d_in, :] = h * mid[k - 1]

    x = x_ref[...]
    B_blk = x.shape[0]
    xc = jnp.clip(x, DOMAIN_MIN, DOMAIN_MAX)
    t = (xc - DOMAIN_MIN) * (1.0 / h)
    idx = jnp.clip(jnp.floor(t).astype(jnp.int32), 0, K - 2)
    u = t - idx.astype(x.dtype)
    u2 = u * u
    u3 = u2 * u
    h00 = 2.0 * u3 - 3.0 * u2 + 1.0
    h10 = u3 - 2.0 * u2 + u
    h01 = -2.0 * u3 + 3.0 * u2
    h11 = u3 - u2

    idx_t = jnp.concatenate([idx] * K, axis=1)
    kk = jax.lax.broadcasted_iota(jnp.int32, (B_blk, K * d_in), 1) // d_in
    at0 = idx_t == kk
    at1 = idx_t == (kk - 1)
    zero = jnp.zeros_like(idx_t, dtype=x.dtype)
    wy = (jnp.where(at0, jnp.concatenate([h00] * K, axis=1), zero)
          + jnp.where(at1, jnp.concatenate([h01] * K, axis=1), zero))
    wm = (jnp.where(at0, jnp.concatenate([h10] * K, axis=1), zero)
          + jnp.where(at1, jnp.concatenate([h11] * K, axis=1), zero))
    w = jnp.concatenate([wy, wm], axis=1)

    acc = jax.lax.dot_general(
        w, a_ref[...],
        dimension_numbers=(((1,), (0,)), ((), ())),
        preferred_element_type=jnp.float32,
    )
    out_ref[...] = acc + bias_ref[...][None, :]


def _tc_matmul_call(xs, yt, bias, *, K, d_in, d_out, h, B_blk=512):
    B = xs.shape[0]
    grid = (B // B_blk,)
    return pl.pallas_call(
        functools.partial(_tc_matmul_kernel, K=K, d_in=d_in, d_out=d_out, h=h),
        grid=grid,
        in_specs=[
            pl.BlockSpec((B_blk, d_in), lambda b: (b, 0)),
            pl.BlockSpec((K, d_in, d_out), lambda b: (0, 0, 0)),
            pl.BlockSpec((d_out,), lambda b: (0,)),
        ],
        out_specs=pl.BlockSpec((B_blk, d_out), lambda b: (b, 0)),
        out_shape=jax.ShapeDtypeStruct((B, d_out), xs.dtype),
        scratch_shapes=[pltpu.VMEM((2 * K * d_in, d_out), jnp.float32)],
    )(xs, yt, bias)


_B_SC = 4096


def kernel(x, y, bias):
    B, d_in = x.shape
    d_out = y.shape[1]
    K = y.shape[2]
    h = (DOMAIN_MAX - DOMAIN_MIN) / (K - 1)
    yt = jnp.transpose(y, (2, 0, 1))

    b_sc = _B_SC
    parts = []
    if b_sc < B:
        parts.append(_tc_matmul_call(x[:B - b_sc], yt, bias,
                                     K=K, d_in=d_in, d_out=d_out, h=h))
    if b_sc > 0:
        ty, tm = _build_tables(yt, K, d_in, d_out, h)
        parts.append(_sc_interp_call(x[B - b_sc:], ty, tm, bias,
                                     K=K, d_in=d_in, d_out=d_out, h=h))
    if len(parts) == 1:
        return parts[0]
    return jnp.concatenate(parts, axis=0)


# speedup vs baseline: 7.1808x; 7.1808x over previous
import functools

import jax
import jax.numpy as jnp
from jax import lax
from jax.experimental import pallas as pl
from jax.experimental.pallas import tpu as pltpu
from jax.experimental.pallas import tpu_sc as plsc

DOMAIN_MIN = -2.0
DOMAIN_MAX = 2.0


def _slopes_from_yt(yt, h, K):
    delta = (yt[1:] - yt[:-1]) * (1.0 / h)
    d0 = delta[:-1]
    d1 = delta[1:]
    same = (d0 * d1) > 0
    denom = d0 + d1
    small = jnp.abs(denom) < 1e-12
    safe_denom = jnp.where(small, jnp.ones_like(denom), denom)
    hm = 2.0 * d0 * d1 / safe_denom
    hm = jnp.where(small, jnp.zeros_like(hm), hm)
    mid = jnp.where(same, hm, jnp.zeros_like(hm))
    m0 = (3.0 * delta[0] - delta[1]) / 2.0
    mN = (3.0 * delta[-1] - delta[-2]) / 2.0
    m0 = jnp.where(m0 * delta[0] <= 0, jnp.zeros_like(m0), m0)
    mN = jnp.where(mN * delta[-1] <= 0, jnp.zeros_like(mN), mN)
    cond0 = (delta[0] * delta[1] < 0) & (jnp.abs(m0) > jnp.abs(3.0 * delta[0]))
    m0 = jnp.where(cond0, 3.0 * delta[0], m0)
    condN = (delta[-1] * delta[-2] < 0) & (jnp.abs(mN) > jnp.abs(3.0 * delta[-1]))
    mN = jnp.where(condN, 3.0 * delta[-1], mN)
    return m0, mid, mN


def _builder_kernel(yt_ref, ty_ref, tm_ref, *, K, h, oh):
    yt = yt_ref[...]
    m0, mid, mN = _slopes_from_yt(yt, h, K)
    mks = [h * m0] + [h * mid[k - 1] for k in range(1, K - 1)] + [h * mN]
    for s in range(2):
        sl = slice(s * oh, (s + 1) * oh)
        for k in range(K):
            c = (k % 4) * oh
            ty_ref[s, :, k // 4, c:c + oh] = yt[k][:, sl]
            tm_ref[s, :, k // 4, c:c + oh] = mks[k][:, sl]


def _build_tables(yt, K, d_in, d_out, h):
    oh = d_out // 2
    return pl.pallas_call(
        functools.partial(_builder_kernel, K=K, h=h, oh=oh),
        out_shape=(
            jax.ShapeDtypeStruct((2, d_in, K // 4, 4 * oh), jnp.float32),
            jax.ShapeDtypeStruct((2, d_in, K // 4, 4 * oh), jnp.float32),
        ),
    )(yt)


def _sc_interp_call(xs, ty, tm, bias, *, K, d_in, d_out, h):
    B_sc = xs.shape[0]
    NC, NS = 2, 16
    NW = NC * NS
    n_bchunk = NW // 2
    bc = B_sc // n_bchunk
    oh = d_out // 2
    inv_h = 1.0 / h
    nrow = d_in * K // 4
    mesh = plsc.VectorSubcoreMesh(core_axis_name="c", subcore_axis_name="s")

    x2 = xs.reshape(B_sc // 2, 2 * d_in)
    ty2 = ty.reshape(2, nrow, 128)
    tm2 = tm.reshape(2, nrow, 128)

    @functools.partial(
        pl.kernel,
        mesh=mesh,
        out_type=jax.ShapeDtypeStruct((2, B_sc // 4, 4 * oh), jnp.float32),
        scratch_types=[
            pltpu.VMEM((bc // 2, 2 * d_in), jnp.float32),
            pltpu.VMEM((nrow, 128), jnp.float32),
            pltpu.VMEM((nrow, 128), jnp.float32),
            pltpu.VMEM((oh,), jnp.float32),
            pltpu.VMEM((bc // 4, 4 * oh), jnp.float32),
        ],
    )
    def sc_kernel(x_hbm, ty_hbm, tm_hbm, bias_hbm, out_hbm,
                  x_v, ty_v, tm_v, bias_v, out_v):
        wid = lax.axis_index("s") * NC + lax.axis_index("c")
        bchunk = wid // 2
        ohalf = wid % 2
        pltpu.sync_copy(x_hbm.at[pl.ds(bchunk * (bc // 2), bc // 2), :], x_v)
        pltpu.sync_copy(ty_hbm.at[ohalf], ty_v)
        pltpu.sync_copy(tm_hbm.at[ohalf], tm_v)
        pltpu.sync_copy(bias_hbm.at[ohalf], bias_v)

        def sample_body(b, _):
            aa = bias_v[pl.ds(0, 16)]
            ab = bias_v[pl.ds(16, 16)]
            xrow = b >> 1
            xcol = (b & 1) * d_in
            for blk in range(d_in // 16):
                xv = x_v[xrow, pl.ds(xcol + blk * 16, 16)]
                xc = jnp.minimum(jnp.maximum(xv, DOMAIN_MIN), DOMAIN_MAX)
                t = (xc - DOMAIN_MIN) * inv_h
                idx = jnp.minimum(t.astype(jnp.int32), K - 2)
                u = t - idx.astype(jnp.float32)
                u2 = u * u
                u3 = u2 * u
                h00v = 2.0 * u3 - 3.0 * u2 + 1.0
                h10v = u3 - 2.0 * u2 + u
                h01v = -2.0 * u3 + 3.0 * u2
                h11v = u3 - u2
                for l in range(16):
                    i = blk * 16 + l
                    k0 = idx[l]
                    k1 = k0 + 1
                    h00 = h00v[l]
                    h10 = h10v[l]
                    h01 = h01v[l]
                    h11 = h11v[l]
                    r0 = i * 4 + (k0 >> 2)
                    c0 = (k0 & 3) * oh
                    r1 = i * 4 + (k1 >> 2)
                    c1 = (k1 & 3) * oh
                    y0a = ty_v[r0, pl.ds(c0, 16)]
                    y0b = ty_v[r0, pl.ds(c0 + 16, 16)]
                    y1a = ty_v[r1, pl.ds(c1, 16)]
                    y1b = ty_v[r1, pl.ds(c1 + 16, 16)]
                    m0a = tm_v[r0, pl.ds(c0, 16)]
                    m0b = tm_v[r0, pl.ds(c0 + 16, 16)]
                    m1a = tm_v[r1, pl.ds(c1, 16)]
                    m1b = tm_v[r1, pl.ds(c1 + 16, 16)]
                    aa = aa + h00 * y0a + h01 * y1a + h10 * m0a + h11 * m1a
                    ab = ab + h00 * y0b + h01 * y1b + h10 * m0b + h11 * m1b
            orow = b >> 2
            ocol = (b & 3) * oh
            out_v[orow, pl.ds(ocol, 16)] = aa
            out_v[orow, pl.ds(ocol + 16, 16)] = ab
            return 0

        lax.fori_loop(0, bc, sample_body, 0)
        pltpu.sync_copy(out_v, out_hbm.at[ohalf, pl.ds(bchunk * (bc // 4), bc // 4), :])

    out2 = sc_kernel(x2, ty2, tm2, bias.reshape(2, oh))
    return jnp.concatenate(
        [out2[0].reshape(B_sc, oh), out2[1].reshape(B_sc, oh)], axis=1)


def _tc_matmul_kernel(x_ref, yt_ref, bias_ref, out_ref, a_ref, *, K, d_in, d_out, h):
    @pl.when(pl.program_id(0) == 0)
    def _build_table():
        yt = yt_ref[...]
        m0, mid, mN = _slopes_from_yt(yt, h, K)
        a_ref[0:d_in, :] = yt[0]
        a_ref[(K - 1) * d_in:K * d_in, :] = yt[K - 1]
        a_ref[K * d_in:(K + 1) * d_in, :] = h * m0
        a_ref[(2 * K - 1) * d_in:2 * K * d_in, :] = h * mN
        for k in range(1, K - 1):
            a_ref[k * d_in:(k + 1) * d_in, :] = yt[k]
            a_ref[(K + k) * d_in:(K + k + 1) * d_in, :] = h * mid[k - 1]

    x = x_ref[...]
    B_blk = x.shape[0]
    xc = jnp.clip(x, DOMAIN_MIN, DOMAIN_MAX)
    t = (xc - DOMAIN_MIN) * (1.0 / h)
    idx = jnp.clip(jnp.floor(t).astype(jnp.int32), 0, K - 2)
    u = t - idx.astype(x.dtype)
    u2 = u * u
    u3 = u2 * u
    h00 = 2.0 * u3 - 3.0 * u2 + 1.0
    h10 = u3 - 2.0 * u2 + u
    h01 = -2.0 * u3 + 3.0 * u2
    h11 = u3 - u2

    idx_t = jnp.concatenate([idx] * K, axis=1)
    kk = jax.lax.broadcasted_iota(jnp.int32, (B_blk, K * d_in), 1) // d_in
    at0 = idx_t == kk
    at1 = idx_t == (kk - 1)
    zero = jnp.zeros_like(idx_t, dtype=x.dtype)
    wy = (jnp.where(at0, jnp.concatenate([h00] * K, axis=1), zero)
          + jnp.where(at1, jnp.concatenate([h01] * K, axis=1), zero))
    wm = (jnp.where(at0, jnp.concatenate([h10] * K, axis=1), zero)
          + jnp.where(at1, jnp.concatenate([h11] * K, axis=1), zero))
    w = jnp.concatenate([wy, wm], axis=1)

    acc = jax.lax.dot_general(
        w, a_ref[...],
        dimension_numbers=(((1,), (0,)), ((), ())),
        preferred_element_type=jnp.float32,
    )
    out_ref[...] = acc + bias_ref[...][None, :]


def _tc_matmul_call(xs, yt, bias, *, K, d_in, d_out, h, B_blk=1024):
    B = xs.shape[0]
    grid = (B // B_blk,)
    return pl.pallas_call(
        functools.partial(_tc_matmul_kernel, K=K, d_in=d_in, d_out=d_out, h=h),
        grid=grid,
        in_specs=[
            pl.BlockSpec((B_blk, d_in), lambda b: (b, 0)),
            pl.BlockSpec((K, d_in, d_out), lambda b: (0, 0, 0)),
            pl.BlockSpec((d_out,), lambda b: (0,)),
        ],
        out_specs=pl.BlockSpec((B_blk, d_out), lambda b: (b, 0)),
        out_shape=jax.ShapeDtypeStruct((B, d_out), xs.dtype),
        scratch_shapes=[pltpu.VMEM((2 * K * d_in, d_out), jnp.float32)],
    )(xs, yt, bias)


_B_SC = 0


def kernel(x, y, bias):
    B, d_in = x.shape
    d_out = y.shape[1]
    K = y.shape[2]
    h = (DOMAIN_MAX - DOMAIN_MIN) / (K - 1)
    yt = jnp.transpose(y, (2, 0, 1))

    b_sc = _B_SC
    parts = []
    if b_sc < B:
        parts.append(_tc_matmul_call(x[:B - b_sc], yt, bias,
                                     K=K, d_in=d_in, d_out=d_out, h=h))
    if b_sc > 0:
        ty, tm = _build_tables(yt, K, d_in, d_out, h)
        parts.append(_sc_interp_call(x[B - b_sc:], ty, tm, bias,
                                     K=K, d_in=d_in, d_out=d_out, h=h))
    if len(parts) == 1:
        return parts[0]
    return jnp.concatenate(parts, axis=0)


# speedup vs baseline: 7.5253x; 1.0480x over previous
import functools

import jax
import jax.numpy as jnp
from jax import lax
from jax.experimental import pallas as pl
from jax.experimental.pallas import tpu as pltpu
from jax.experimental.pallas import tpu_sc as plsc

DOMAIN_MIN = -2.0
DOMAIN_MAX = 2.0


def _slopes_from_yt(yt, h, K):
    delta = (yt[1:] - yt[:-1]) * (1.0 / h)
    d0 = delta[:-1]
    d1 = delta[1:]
    same = (d0 * d1) > 0
    denom = d0 + d1
    small = jnp.abs(denom) < 1e-12
    safe_denom = jnp.where(small, jnp.ones_like(denom), denom)
    hm = 2.0 * d0 * d1 / safe_denom
    hm = jnp.where(small, jnp.zeros_like(hm), hm)
    mid = jnp.where(same, hm, jnp.zeros_like(hm))
    m0 = (3.0 * delta[0] - delta[1]) / 2.0
    mN = (3.0 * delta[-1] - delta[-2]) / 2.0
    m0 = jnp.where(m0 * delta[0] <= 0, jnp.zeros_like(m0), m0)
    mN = jnp.where(mN * delta[-1] <= 0, jnp.zeros_like(mN), mN)
    cond0 = (delta[0] * delta[1] < 0) & (jnp.abs(m0) > jnp.abs(3.0 * delta[0]))
    m0 = jnp.where(cond0, 3.0 * delta[0], m0)
    condN = (delta[-1] * delta[-2] < 0) & (jnp.abs(mN) > jnp.abs(3.0 * delta[-1]))
    mN = jnp.where(condN, 3.0 * delta[-1], mN)
    return m0, mid, mN


def _builder_kernel(yt_ref, ty_ref, tm_ref, *, K, h, oh):
    yt = yt_ref[...]
    m0, mid, mN = _slopes_from_yt(yt, h, K)
    mks = [h * m0] + [h * mid[k - 1] for k in range(1, K - 1)] + [h * mN]
    for s in range(2):
        sl = slice(s * oh, (s + 1) * oh)
        for k in range(K):
            c = (k % 4) * oh
            ty_ref[s, :, k // 4, c:c + oh] = yt[k][:, sl]
            tm_ref[s, :, k // 4, c:c + oh] = mks[k][:, sl]


def _build_tables(yt, K, d_in, d_out, h):
    oh = d_out // 2
    return pl.pallas_call(
        functools.partial(_builder_kernel, K=K, h=h, oh=oh),
        out_shape=(
            jax.ShapeDtypeStruct((2, d_in, K // 4, 4 * oh), jnp.float32),
            jax.ShapeDtypeStruct((2, d_in, K // 4, 4 * oh), jnp.float32),
        ),
    )(yt)


def _sc_interp_call(xs, ty, tm, bias, *, K, d_in, d_out, h):
    B_sc = xs.shape[0]
    NC, NS = 2, 16
    NW = NC * NS
    n_bchunk = NW // 2
    bc = B_sc // n_bchunk
    oh = d_out // 2
    inv_h = 1.0 / h
    nrow = d_in * K // 4
    mesh = plsc.VectorSubcoreMesh(core_axis_name="c", subcore_axis_name="s")

    x2 = xs.reshape(B_sc // 2, 2 * d_in)
    ty2 = ty.reshape(2, nrow, 128)
    tm2 = tm.reshape(2, nrow, 128)

    @functools.partial(
        pl.kernel,
        mesh=mesh,
        out_type=jax.ShapeDtypeStruct((2, B_sc // 4, 4 * oh), jnp.float32),
        scratch_types=[
            pltpu.VMEM((bc // 2, 2 * d_in), jnp.float32),
            pltpu.VMEM((nrow, 128), jnp.float32),
            pltpu.VMEM((nrow, 128), jnp.float32),
            pltpu.VMEM((oh,), jnp.float32),
            pltpu.VMEM((bc // 4, 4 * oh), jnp.float32),
        ],
    )
    def sc_kernel(x_hbm, ty_hbm, tm_hbm, bias_hbm, out_hbm,
                  x_v, ty_v, tm_v, bias_v, out_v):
        wid = lax.axis_index("s") * NC + lax.axis_index("c")
        bchunk = wid // 2
        ohalf = wid % 2
        pltpu.sync_copy(x_hbm.at[pl.ds(bchunk * (bc // 2), bc // 2), :], x_v)
        pltpu.sync_copy(ty_hbm.at[ohalf], ty_v)
        pltpu.sync_copy(tm_hbm.at[ohalf], tm_v)
        pltpu.sync_copy(bias_hbm.at[ohalf], bias_v)

        def sample_body(b, _):
            aa = bias_v[pl.ds(0, 16)]
            ab = bias_v[pl.ds(16, 16)]
            xrow = b >> 1
            xcol = (b & 1) * d_in
            for blk in range(d_in // 16):
                xv = x_v[xrow, pl.ds(xcol + blk * 16, 16)]
                xc = jnp.minimum(jnp.maximum(xv, DOMAIN_MIN), DOMAIN_MAX)
                t = (xc - DOMAIN_MIN) * inv_h
                idx = jnp.minimum(t.astype(jnp.int32), K - 2)
                u = t - idx.astype(jnp.float32)
                u2 = u * u
                u3 = u2 * u
                h00v = 2.0 * u3 - 3.0 * u2 + 1.0
                h10v = u3 - 2.0 * u2 + u
                h01v = -2.0 * u3 + 3.0 * u2
                h11v = u3 - u2
                for l in range(16):
                    i = blk * 16 + l
                    k0 = idx[l]
                    k1 = k0 + 1
                    h00 = h00v[l]
                    h10 = h10v[l]
                    h01 = h01v[l]
                    h11 = h11v[l]
                    r0 = i * 4 + (k0 >> 2)
                    c0 = (k0 & 3) * oh
                    r1 = i * 4 + (k1 >> 2)
                    c1 = (k1 & 3) * oh
                    y0a = ty_v[r0, pl.ds(c0, 16)]
                    y0b = ty_v[r0, pl.ds(c0 + 16, 16)]
                    y1a = ty_v[r1, pl.ds(c1, 16)]
                    y1b = ty_v[r1, pl.ds(c1 + 16, 16)]
                    m0a = tm_v[r0, pl.ds(c0, 16)]
                    m0b = tm_v[r0, pl.ds(c0 + 16, 16)]
                    m1a = tm_v[r1, pl.ds(c1, 16)]
                    m1b = tm_v[r1, pl.ds(c1 + 16, 16)]
                    aa = aa + h00 * y0a + h01 * y1a + h10 * m0a + h11 * m1a
                    ab = ab + h00 * y0b + h01 * y1b + h10 * m0b + h11 * m1b
            orow = b >> 2
            ocol = (b & 3) * oh
            out_v[orow, pl.ds(ocol, 16)] = aa
            out_v[orow, pl.ds(ocol + 16, 16)] = ab
            return 0

        lax.fori_loop(0, bc, sample_body, 0)
        pltpu.sync_copy(out_v, out_hbm.at[ohalf, pl.ds(bchunk * (bc // 4), bc // 4), :])

    out2 = sc_kernel(x2, ty2, tm2, bias.reshape(2, oh))
    return jnp.concatenate(
        [out2[0].reshape(B_sc, oh), out2[1].reshape(B_sc, oh)], axis=1)


def _tc_matmul_kernel(x_ref, yt_ref, bias_ref, out_ref, a_ref, *, K, d_in, d_out, h):
    @pl.when(pl.program_id(0) == 0)
    def _build_table():
        yt = yt_ref[...]
        m0, mid, mN = _slopes_from_yt(yt, h, K)
        bf = jnp.bfloat16
        a_ref[0:d_in, :] = yt[0].astype(bf)
        a_ref[(K - 1) * d_in:K * d_in, :] = yt[K - 1].astype(bf)
        a_ref[K * d_in:(K + 1) * d_in, :] = (h * m0).astype(bf)
        a_ref[(2 * K - 1) * d_in:2 * K * d_in, :] = (h * mN).astype(bf)
        for k in range(1, K - 1):
            a_ref[k * d_in:(k + 1) * d_in, :] = yt[k].astype(bf)
            a_ref[(K + k) * d_in:(K + k + 1) * d_in, :] = (h * mid[k - 1]).astype(bf)

    x = x_ref[...]
    B_blk = x.shape[0]
    xc = jnp.clip(x, DOMAIN_MIN, DOMAIN_MAX)
    t = (xc - DOMAIN_MIN) * (1.0 / h)
    idx = jnp.clip(jnp.floor(t).astype(jnp.int32), 0, K - 2)
    u = t - idx.astype(x.dtype)
    u2 = u * u
    u3 = u2 * u
    bf = jnp.bfloat16
    h00 = (2.0 * u3 - 3.0 * u2 + 1.0).astype(bf)
    h10 = (u3 - 2.0 * u2 + u).astype(bf)
    h01 = (-2.0 * u3 + 3.0 * u2).astype(bf)
    h11 = (u3 - u2).astype(bf)
    idx_bf = idx.astype(bf)

    idx_t = jnp.concatenate([idx_bf] * K, axis=1)
    kk = (jax.lax.broadcasted_iota(jnp.int32, (B_blk, K * d_in), 1)
          // d_in).astype(bf)
    at0 = idx_t == kk
    at1 = idx_t == (kk - jnp.ones((), bf))
    wy = jnp.where(at0, jnp.concatenate([h00] * K, axis=1),
                   jnp.where(at1, jnp.concatenate([h01] * K, axis=1),
                             jnp.zeros((), bf)))
    wm = jnp.where(at0, jnp.concatenate([h10] * K, axis=1),
                   jnp.where(at1, jnp.concatenate([h11] * K, axis=1),
                             jnp.zeros((), bf)))
    w = jnp.concatenate([wy, wm], axis=1)

    acc = jax.lax.dot_general(
        w, a_ref[...],
        dimension_numbers=(((1,), (0,)), ((), ())),
        preferred_element_type=jnp.float32,
    )
    out_ref[...] = acc + bias_ref[...][None, :]


def _tc_matmul_call(xs, yt, bias, *, K, d_in, d_out, h, B_blk=1024):
    B = xs.shape[0]
    grid = (B // B_blk,)
    return pl.pallas_call(
        functools.partial(_tc_matmul_kernel, K=K, d_in=d_in, d_out=d_out, h=h),
        grid=grid,
        in_specs=[
            pl.BlockSpec((B_blk, d_in), lambda b: (b, 0)),
            pl.BlockSpec((K, d_in, d_out), lambda b: (0, 0, 0)),
            pl.BlockSpec((d_out,), lambda b: (0,)),
        ],
        out_specs=pl.BlockSpec((B_blk, d_out), lambda b: (b, 0)),
        out_shape=jax.ShapeDtypeStruct((B, d_out), xs.dtype),
        scratch_shapes=[pltpu.VMEM((2 * K * d_in, d_out), jnp.bfloat16)],
    )(xs, yt, bias)


_B_SC = 0


def kernel(x, y, bias):
    B, d_in = x.shape
    d_out = y.shape[1]
    K = y.shape[2]
    h = (DOMAIN_MAX - DOMAIN_MIN) / (K - 1)
    yt = jnp.transpose(y, (2, 0, 1))

    b_sc = _B_SC
    parts = []
    if b_sc < B:
        parts.append(_tc_matmul_call(x[:B - b_sc], yt, bias,
                                     K=K, d_in=d_in, d_out=d_out, h=h))
    if b_sc > 0:
        ty, tm = _build_tables(yt, K, d_in, d_out, h)
        parts.append(_sc_interp_call(x[B - b_sc:], ty, tm, bias,
                                     K=K, d_in=d_in, d_out=d_out, h=h))
    if len(parts) == 1:
        return parts[0]
    return jnp.concatenate(parts, axis=0)
